# baseline (speedup 1.0000x reference)
import jax
import jax.numpy as jnp
from jax import lax
from jax.experimental import pallas as pl
from jax.experimental.pallas import tpu as pltpu

TAU_WORD = 0.8
INV_TAU = 1.0 / TAU_WORD
RB = 4


def _body(tgt_ref, msk_ref, logp_ref, sim_hbm, ml_ref, out_ref, msk_sum_ref,
          sim_buf, sems):
    i = pl.program_id(0)
    n = pl.num_programs(0)
    l = logp_ref.shape[1]
    rows = sim_buf.shape[1]

    def issue(step, slot):
        for k in range(RB):
            for j in range(l):
                t = tgt_ref[step * RB + k, j]
                pltpu.make_async_copy(
                    sim_hbm.at[pl.ds(t, 1), :],
                    sim_buf.at[slot, pl.ds(k * l + j, 1), :],
                    sems.at[slot, k * l + j],
                ).start()

    @pl.when(i == 0)
    def _prologue():
        ml_ref[0, 0] = 0.0
        out_ref[0, 0] = 0.0
        msk_sum_ref[0, 0] = 0.0
        issue(0, 0)

    @pl.when(i + 1 < n)
    def _prefetch():
        issue(i + 1, (i + 1) % 2)

    slot = i % 2
    for r in range(rows):
        pltpu.make_async_copy(
            sim_hbm.at[pl.ds(0, 1), :],
            sim_buf.at[slot, pl.ds(r, 1), :],
            sems.at[slot, r],
        ).wait()

    ml_acc = 0.0
    out_acc = 0.0
    msk_acc = 0.0
    for k in range(RB):
        sim_blk = sim_buf[slot, pl.ds(k * l, l)]
        logp_blk = logp_ref[k]
        e = jnp.exp(sim_blk * INV_TAU)
        s = jnp.sum(e, axis=1)
        d = jnp.sum(logp_blk * e, axis=1)

        tvals = jnp.stack([tgt_ref[i * RB + k, j] for j in range(l)])
        mvals = jnp.stack([msk_ref[i * RB + k, j] for j in range(l)])
        col = lax.broadcasted_iota(jnp.int32, logp_blk.shape, 1)
        lp_t = jnp.sum(jnp.where(col == tvals[:, None], logp_blk, 0.0),
                       axis=1)

        ml_acc += jnp.sum(-mvals * lp_t)
        out_acc += jnp.sum(-mvals * d / s)
        msk_acc += jnp.sum(mvals)

    ml_ref[0, 0] += ml_acc
    out_ref[0, 0] += out_acc
    msk_sum_ref[0, 0] += msk_acc

    @pl.when(i == n - 1)
    def _fin():
        denom = msk_sum_ref[0, 0]
        ml_ref[0, 0] = ml_ref[0, 0] / denom
        out_ref[0, 0] = out_ref[0, 0] / denom


@jax.jit
def _run(logp, tgt, msk, sim_matrix):
    b, l, v = logp.shape
    grid_spec = pltpu.PrefetchScalarGridSpec(
        num_scalar_prefetch=2,
        grid=(b // RB,),
        in_specs=[
            pl.BlockSpec((RB, l, v), lambda i, tgt, msk: (i, 0, 0)),
            pl.BlockSpec(memory_space=pl.ANY),
        ],
        out_specs=[
            pl.BlockSpec(memory_space=pltpu.SMEM),
            pl.BlockSpec(memory_space=pltpu.SMEM),
            pl.BlockSpec(memory_space=pltpu.SMEM),
        ],
        scratch_shapes=[
            pltpu.VMEM((2, RB * l, v), jnp.float32),
            pltpu.SemaphoreType.DMA((2, RB * l)),
        ],
    )
    ml, out, _ = pl.pallas_call(
        _body,
        grid_spec=grid_spec,
        out_shape=[
            jax.ShapeDtypeStruct((1, 1), jnp.float32),
            jax.ShapeDtypeStruct((1, 1), jnp.float32),
            jax.ShapeDtypeStruct((1, 1), jnp.float32),
        ],
        compiler_params=pltpu.CompilerParams(
            dimension_semantics=("arbitrary",),
        ),
    )(tgt, msk, logp, sim_matrix)
    return ml[0, 0], out[0, 0]


def kernel(logp, target, mask, sim_matrix):
    tgt = target.astype(jnp.int32)
    msk = mask.astype(jnp.float32)
    return _run(logp, tgt, msk, sim_matrix)


# speedup vs baseline: 1.0014x; 1.0014x over previous
import jax
import jax.numpy as jnp
from jax import lax
from jax.experimental import pallas as pl
from jax.experimental.pallas import tpu as pltpu

TAU_WORD = 0.8
INV_TAU = 1.0 / TAU_WORD
RB = 8


def _body(tgt_ref, msk_ref, logp_ref, sim_hbm, ml_ref, out_ref, msk_sum_ref,
          sim_buf, sems):
    i = pl.program_id(0)
    n = pl.num_programs(0)
    l = logp_ref.shape[1]
    rows = sim_buf.shape[1]

    def issue(step, slot):
        for k in range(RB):
            for j in range(l):
                t = tgt_ref[step * RB + k, j]
                pltpu.make_async_copy(
                    sim_hbm.at[pl.ds(t, 1), :],
                    sim_buf.at[slot, pl.ds(k * l + j, 1), :],
                    sems.at[slot, k * l + j],
                ).start()

    @pl.when(i == 0)
    def _prologue():
        ml_ref[0, 0] = 0.0
        out_ref[0, 0] = 0.0
        msk_sum_ref[0, 0] = 0.0
        issue(0, 0)

    @pl.when(i + 1 < n)
    def _prefetch():
        issue(i + 1, (i + 1) % 2)

    slot = i % 2
    for r in range(rows):
        pltpu.make_async_copy(
            sim_hbm.at[pl.ds(0, 1), :],
            sim_buf.at[slot, pl.ds(r, 1), :],
            sems.at[slot, r],
        ).wait()

    ml_acc = 0.0
    out_acc = 0.0
    msk_acc = 0.0
    for k in range(RB):
        sim_blk = sim_buf[slot, pl.ds(k * l, l)]
        logp_blk = logp_ref[k]
        e = jnp.exp(sim_blk * INV_TAU)
        s = jnp.sum(e, axis=1)
        d = jnp.sum(logp_blk * e, axis=1)

        tvals = jnp.stack([tgt_ref[i * RB + k, j] for j in range(l)])
        mvals = jnp.stack([msk_ref[i * RB + k, j] for j in range(l)])
        col = lax.broadcasted_iota(jnp.int32, logp_blk.shape, 1)
        lp_t = jnp.sum(jnp.where(col == tvals[:, None], logp_blk, 0.0),
                       axis=1)

        ml_acc += jnp.sum(-mvals * lp_t)
        out_acc += jnp.sum(-mvals * d / s)
        msk_acc += jnp.sum(mvals)

    ml_ref[0, 0] += ml_acc
    out_ref[0, 0] += out_acc
    msk_sum_ref[0, 0] += msk_acc

    @pl.when(i == n - 1)
    def _fin():
        denom = msk_sum_ref[0, 0]
        ml_ref[0, 0] = ml_ref[0, 0] / denom
        out_ref[0, 0] = out_ref[0, 0] / denom


@jax.jit
def _run(logp, tgt, msk, sim_matrix):
    b, l, v = logp.shape
    grid_spec = pltpu.PrefetchScalarGridSpec(
        num_scalar_prefetch=2,
        grid=(b // RB,),
        in_specs=[
            pl.BlockSpec((RB, l, v), lambda i, tgt, msk: (i, 0, 0)),
            pl.BlockSpec(memory_space=pl.ANY),
        ],
        out_specs=[
            pl.BlockSpec(memory_space=pltpu.SMEM),
            pl.BlockSpec(memory_space=pltpu.SMEM),
            pl.BlockSpec(memory_space=pltpu.SMEM),
        ],
        scratch_shapes=[
            pltpu.VMEM((2, RB * l, v), jnp.float32),
            pltpu.SemaphoreType.DMA((2, RB * l)),
        ],
    )
    ml, out, _ = pl.pallas_call(
        _body,
        grid_spec=grid_spec,
        out_shape=[
            jax.ShapeDtypeStruct((1, 1), jnp.float32),
            jax.ShapeDtypeStruct((1, 1), jnp.float32),
            jax.ShapeDtypeStruct((1, 1), jnp.float32),
        ],
        compiler_params=pltpu.CompilerParams(
            dimension_semantics=("arbitrary",),
        ),
    )(tgt, msk, logp, sim_matrix)
    return ml[0, 0], out[0, 0]


def kernel(logp, target, mask, sim_matrix):
    tgt = target.astype(jnp.int32)
    msk = mask.astype(jnp.float32)
    return _run(logp, tgt, msk, sim_matrix)


# speedup vs baseline: 1.0315x; 1.0301x over previous
import jax
import jax.numpy as jnp
from jax import lax
from jax.experimental import pallas as pl
from jax.experimental.pallas import tpu as pltpu

TAU_WORD = 0.8
INV_TAU = 1.0 / TAU_WORD
RB = 4


def _body(tgt_ref, msk_ref, logp_ref, sim_hbm, ml_ref, out_ref, msk_sum_ref,
          sim_buf, sems):
    i = pl.program_id(0)
    n = pl.num_programs(0)
    l = logp_ref.shape[1]
    rows = sim_buf.shape[1]

    def issue(step, slot):
        for k in range(RB):
            for j in range(l):
                t = tgt_ref[step * RB + k, j]
                pltpu.make_async_copy(
                    sim_hbm.at[pl.ds(t, 1), :],
                    sim_buf.at[slot, pl.ds(k * l + j, 1), :],
                    sems.at[slot, k * l + j],
                ).start()

    @pl.when(i == 0)
    def _prologue():
        ml_ref[0, 0] = 0.0
        out_ref[0, 0] = 0.0
        msk_sum_ref[0, 0] = 0.0
        issue(0, 0)
        issue(1, 1)

    @pl.when(i + 2 < n)
    def _prefetch():
        issue(i + 2, lax.rem(i + 2, 3))

    slot = lax.rem(i, 3)
    for r in range(rows):
        pltpu.make_async_copy(
            sim_hbm.at[pl.ds(0, 1), :],
            sim_buf.at[slot, pl.ds(r, 1), :],
            sems.at[slot, r],
        ).wait()

    ml_acc = 0.0
    out_acc = 0.0
    msk_acc = 0.0
    for k in range(RB):
        sim_blk = sim_buf[slot, pl.ds(k * l, l)]
        logp_blk = logp_ref[k]
        e = jnp.exp(sim_blk * INV_TAU)
        s = jnp.sum(e, axis=1)
        d = jnp.sum(logp_blk * e, axis=1)

        tvals = jnp.stack([tgt_ref[i * RB + k, j] for j in range(l)])
        mvals = jnp.stack([msk_ref[i * RB + k, j] for j in range(l)])
        col = lax.broadcasted_iota(jnp.int32, logp_blk.shape, 1)
        lp_t = jnp.sum(jnp.where(col == tvals[:, None], logp_blk, 0.0),
                       axis=1)

        ml_acc += jnp.sum(-mvals * lp_t)
        out_acc += jnp.sum(-mvals * d / s)
        msk_acc += jnp.sum(mvals)

    ml_ref[0, 0] += ml_acc
    out_ref[0, 0] += out_acc
    msk_sum_ref[0, 0] += msk_acc

    @pl.when(i == n - 1)
    def _fin():
        denom = msk_sum_ref[0, 0]
        ml_ref[0, 0] = ml_ref[0, 0] / denom
        out_ref[0, 0] = out_ref[0, 0] / denom


@jax.jit
def _run(logp, tgt, msk, sim_matrix):
    b, l, v = logp.shape
    grid_spec = pltpu.PrefetchScalarGridSpec(
        num_scalar_prefetch=2,
        grid=(b // RB,),
        in_specs=[
            pl.BlockSpec((RB, l, v), lambda i, tgt, msk: (i, 0, 0)),
            pl.BlockSpec(memory_space=pl.ANY),
        ],
        out_specs=[
            pl.BlockSpec(memory_space=pltpu.SMEM),
            pl.BlockSpec(memory_space=pltpu.SMEM),
            pl.BlockSpec(memory_space=pltpu.SMEM),
        ],
        scratch_shapes=[
            pltpu.VMEM((3, RB * l, v), jnp.float32),
            pltpu.SemaphoreType.DMA((3, RB * l)),
        ],
    )
    ml, out, _ = pl.pallas_call(
        _body,
        grid_spec=grid_spec,
        out_shape=[
            jax.ShapeDtypeStruct((1, 1), jnp.float32),
            jax.ShapeDtypeStruct((1, 1), jnp.float32),
            jax.ShapeDtypeStruct((1, 1), jnp.float32),
        ],
        compiler_params=pltpu.CompilerParams(
            dimension_semantics=("arbitrary",),
        ),
    )(tgt, msk, logp, sim_matrix)
    return ml[0, 0], out[0, 0]


def kernel(logp, target, mask, sim_matrix):
    tgt = target.astype(jnp.int32)
    msk = mask.astype(jnp.float32)
    return _run(logp, tgt, msk, sim_matrix)
